# Initial kernel scaffold; baseline (speedup 1.0000x reference)
import jax
import jax.numpy as jnp
from jax.experimental import pallas as pl


def kernel(hidden_states, router_weight, gate_up_proj, down_proj, shared_gate_w, shared_up_w, shared_down_w, shared_expert_gate_w):
    raise NotImplementedError("write your pallas kernel here")


# speedup vs baseline: 1.2062x; 1.2062x over previous
import functools

import jax
import jax.numpy as jnp
from jax.experimental import pallas as pl
from jax.experimental.pallas import tpu as pltpu

E = 8
TOP_K = 2
D = 1024
I = 512
IS = 1408

TM = 256


def _moe_kernel(x_ref, rw_ref, gu_ref, dn_ref, sg_ref, su_ref, sd_ref, seg_ref,
                out_ref):
    e = pl.program_id(1)
    x = x_ref[...]

    @pl.when(e == 0)
    def _init():
        g = jax.lax.dot_general(x, sg_ref[...], (((1,), (1,)), ((), ())),
                                preferred_element_type=jnp.float32)
        u = jax.lax.dot_general(x, su_ref[...], (((1,), (1,)), ((), ())),
                                preferred_element_type=jnp.float32)
        h = (g * jax.lax.logistic(g)) * u
        shared = jax.lax.dot_general(h, sd_ref[...], (((1,), (1,)), ((), ())),
                                     preferred_element_type=jnp.float32)
        sgate = jax.lax.logistic(
            jax.lax.dot_general(x, seg_ref[...], (((1,), (1,)), ((), ())),
                                preferred_element_type=jnp.float32))
        out_ref[...] = sgate * shared

    logits = jax.lax.dot_general(x, rw_ref[...], (((1,), (1,)), ((), ())),
                                 preferred_element_type=jnp.float32)
    m1 = jnp.max(logits, axis=-1, keepdims=True)
    i1 = jnp.argmax(logits, axis=-1, keepdims=True)
    eids = jax.lax.broadcasted_iota(jnp.int32, logits.shape, 1)
    masked = jnp.where(eids == i1, -jnp.inf, logits)
    m2 = jnp.max(masked, axis=-1, keepdims=True)
    i2 = jnp.argmax(masked, axis=-1, keepdims=True)
    w1 = jax.lax.logistic(m1 - m2)
    w2 = 1.0 - w1
    we = jnp.where(i1 == e, w1, jnp.where(i2 == e, w2, 0.0))

    gu = jax.lax.dot_general(x, gu_ref[0], (((1,), (1,)), ((), ())),
                             preferred_element_type=jnp.float32)
    g = gu[:, :I]
    u = gu[:, I:]
    h = (g * jax.lax.logistic(g)) * u
    o = jax.lax.dot_general(h, dn_ref[0], (((1,), (1,)), ((), ())),
                            preferred_element_type=jnp.float32)
    out_ref[...] += we * o


@functools.partial(jax.jit, static_argnames=())
def kernel(hidden_states, router_weight, gate_up_proj, down_proj,
           shared_gate_w, shared_up_w, shared_down_w, shared_expert_gate_w):
    b, s, d = hidden_states.shape
    x = hidden_states.reshape(-1, d)
    t = x.shape[0]
    nt = t // TM

    out = pl.pallas_call(
        _moe_kernel,
        grid=(nt, E),
        in_specs=[
            pl.BlockSpec((TM, D), lambda i, e: (i, 0)),
            pl.BlockSpec((E, D), lambda i, e: (0, 0)),
            pl.BlockSpec((1, 2 * I, D), lambda i, e: (e, 0, 0)),
            pl.BlockSpec((1, D, I), lambda i, e: (e, 0, 0)),
            pl.BlockSpec((IS, D), lambda i, e: (0, 0)),
            pl.BlockSpec((IS, D), lambda i, e: (0, 0)),
            pl.BlockSpec((D, IS), lambda i, e: (0, 0)),
            pl.BlockSpec((1, D), lambda i, e: (0, 0)),
        ],
        out_specs=pl.BlockSpec((TM, D), lambda i, e: (i, 0)),
        out_shape=jax.ShapeDtypeStruct((t, d), jnp.float32),
        compiler_params=pltpu.CompilerParams(
            dimension_semantics=("parallel", "arbitrary"),
        ),
    )(x, router_weight, gate_up_proj, down_proj,
      shared_gate_w, shared_up_w, shared_down_w, shared_expert_gate_w)

    return out.reshape(b, s, d)
